# baseline (speedup 1.0000x reference)
import functools

import jax
import jax.numpy as jnp
from jax import lax
from jax.experimental import pallas as pl
from jax.experimental.pallas import tpu as pltpu
from jax.experimental.pallas import tpu_sc as plsc

f32 = jnp.float32
i32 = jnp.int32

_NW = 32
_NSUB = 16
_L = 16


def _ceil_to(x, m):
    return (x + m - 1) // m * m


def _k1_matvec(x, w):
    n, d = x.shape
    blk = 1000
    grid = n // blk

    def body(x_ref, w_ref, o_ref):
        o_ref[...] = jnp.dot(x_ref[...], w_ref[...],
                             preferred_element_type=f32,
                             precision=lax.Precision.HIGHEST)

    return pl.pallas_call(
        body,
        grid=(grid,),
        in_specs=[pl.BlockSpec((blk, d), lambda i: (i, 0)),
                  pl.BlockSpec((d, 8), lambda i: (0, 0))],
        out_specs=pl.BlockSpec((blk, 8), lambda i: (i, 0)),
        out_shape=jax.ShapeDtypeStruct((n, 8), f32),
    )(x, w)


def _k2_scores(a, b, col, row, ew, *, nb, ch, etp):
    mesh = plsc.VectorSubcoreMesh(core_axis_name="c", subcore_axis_name="s")

    @functools.partial(
        pl.kernel,
        out_type=[jax.ShapeDtypeStruct((_NW, nb), f32),
                  jax.ShapeDtypeStruct((_NW, nb), f32),
                  jax.ShapeDtypeStruct((etp,), f32)],
        mesh=mesh,
        scratch_types=[pltpu.VMEM((nb,), f32),
                       pltpu.VMEM((nb,), f32),
                       pltpu.VMEM((ch,), i32),
                       pltpu.VMEM((ch,), i32),
                       pltpu.VMEM((ch,), f32),
                       pltpu.VMEM((ch,), f32),
                       pltpu.VMEM((nb,), f32),
                       pltpu.VMEM((nb,), f32)],
        compiler_params=pltpu.CompilerParams(needs_layout_passes=False),
    )
    def k(a_hbm, b_hbm, col_hbm, row_hbm, ew_hbm, h_hbm, s_hbm, x_hbm,
          a_v, b_v, col_v, row_v, ew_v, x_v, h_v, s_v):
        wid = lax.axis_index("s") * 2 + lax.axis_index("c")
        e0 = pl.multiple_of(wid * ch, 8)
        pltpu.sync_copy(a_hbm, a_v)
        pltpu.sync_copy(b_hbm, b_v)
        pltpu.sync_copy(col_hbm.at[pl.ds(e0, ch)], col_v)
        pltpu.sync_copy(row_hbm.at[pl.ds(e0, ch)], row_v)
        pltpu.sync_copy(ew_hbm.at[pl.ds(e0, ch)], ew_v)

        zeros = jnp.zeros((_L,), f32)

        def zbody(i, _):
            sl = pl.ds(pl.multiple_of(i * _L, _L), _L)
            h_v[sl] = zeros
            s_v[sl] = zeros
            return ()

        lax.fori_loop(0, nb // _L, zbody, (), unroll=4)

        ones = jnp.full((_L,), 1.0, f32)

        def gbody(g, _):
            sl = pl.ds(pl.multiple_of(g * _L, _L), _L)
            c = col_v[sl]
            r = row_v[sl]
            w = plsc.load_gather(a_v, [r]) + plsc.load_gather(b_v, [c])
            w = jnp.where(w >= 0, w, 0.01 * w) + ew_v[sl]
            e = jnp.exp(w)
            x_v[sl] = e
            plsc.addupdate_scatter(h_v, [c], ones)
            plsc.addupdate_scatter(s_v, [c], e)
            return ()

        lax.fori_loop(0, ch // _L, gbody, (), unroll=2)

        pltpu.sync_copy(h_v, h_hbm.at[wid])
        pltpu.sync_copy(s_v, s_hbm.at[wid])
        pltpu.sync_copy(x_v, x_hbm.at[pl.ds(e0, ch)])

    return k(a, b, col, row, ew)


def _k3_offsets(h, s, *, nb):
    binw = nb // _NSUB
    mesh = plsc.VectorSubcoreMesh(core_axis_name="c", subcore_axis_name="s")

    @functools.partial(
        pl.kernel,
        out_type=[jax.ShapeDtypeStruct((_NW, nb), f32),
                  jax.ShapeDtypeStruct((nb,), f32),
                  jax.ShapeDtypeStruct((_NSUB, _L), f32)],
        mesh=mesh,
        scratch_types=[pltpu.VMEM((_NW, binw), f32),
                       pltpu.VMEM((_NW, binw), f32),
                       pltpu.VMEM((_NW, binw), f32),
                       pltpu.VMEM((binw,), f32),
                       pltpu.VMEM((binw,), f32),
                       pltpu.VMEM((binw,), f32),
                       pltpu.VMEM((binw,), f32),
                       pltpu.VMEM((_L,), f32),
                       pltpu.VMEM((_NSUB, _L), f32)],
        compiler_params=pltpu.CompilerParams(needs_layout_passes=False),
    )
    def k(h_hbm, s_hbm, base_hbm, stot_hbm, tot_hbm,
          h_v, s_v, p_v, cnt_v, incl_v, stot_v, tmp_v, c16_v, tv_v):
        cid = lax.axis_index("c")
        sid = lax.axis_index("s")

        @pl.when(cid == 0)
        def _():
            c0 = pl.multiple_of(sid * binw, 8)

            def ld(t, _):
                pltpu.sync_copy(h_hbm.at[t, pl.ds(c0, binw)], h_v.at[t])
                pltpu.sync_copy(s_hbm.at[t, pl.ds(c0, binw)], s_v.at[t])
                return ()

            lax.fori_loop(0, _NW, ld, ())

            zeros = jnp.zeros((_L,), f32)

            def cbody(ci, _):
                sl = pl.ds(pl.multiple_of(ci * _L, _L), _L)

                def tbody(t, carry):
                    acc, sacc = carry
                    p_v[t, sl] = acc
                    return acc + h_v[t, sl], sacc + s_v[t, sl]

                acc, sacc = lax.fori_loop(0, _NW, tbody, (zeros, zeros))
                cnt_v[sl] = acc
                stot_v[sl] = sacc
                return ()

            lax.fori_loop(0, binw // _L, cbody, ())

            last_idx = jnp.full((_L,), _L - 1, i32)

            def csum(ci, carry):
                sl = pl.ds(pl.multiple_of(ci * _L, _L), _L)
                cs = plsc.cumsum(cnt_v[sl]) + carry
                incl_v[sl] = cs
                return cs.at[last_idx].get(mode="promise_in_bounds")

            tot = lax.fori_loop(0, binw // _L, csum, zeros)

            c16_v[...] = tot
            pltpu.sync_copy(c16_v, tot_hbm.at[sid])
            plsc.subcore_barrier()
            pltpu.sync_copy(tot_hbm, tv_v)
            io = lax.iota(i32, _L)
            tvals = plsc.load_gather(tv_v, [io, jnp.zeros((_L,), i32)])
            pfx = plsc.cumsum(tvals) - tvals
            off = pfx.at[jnp.full((_L,), sid, i32)].get(
                mode="promise_in_bounds")

            def wbody(t, _):
                def wci(ci, _):
                    sl = pl.ds(pl.multiple_of(ci * _L, _L), _L)
                    excl = incl_v[sl] + off - cnt_v[sl]
                    tmp_v[sl] = p_v[t, sl] + excl
                    return ()

                lax.fori_loop(0, binw // _L, wci, ())
                pltpu.sync_copy(tmp_v, base_hbm.at[t, pl.ds(c0, binw)])
                return ()

            lax.fori_loop(0, _NW, wbody, ())
            pltpu.sync_copy(stot_v, stot_hbm.at[pl.ds(c0, binw)])

    return k(h, s)


def _k4_scatter(base, stot, col, row, ew, em, expw, kc, fill,
                *, nb, ch, etp, n):
    mesh = plsc.VectorSubcoreMesh(core_axis_name="c", subcore_axis_name="s")

    @functools.partial(
        pl.kernel,
        out_type=[jax.ShapeDtypeStruct((etp,), i32),
                  jax.ShapeDtypeStruct((etp,), i32),
                  jax.ShapeDtypeStruct((etp,), f32),
                  jax.ShapeDtypeStruct((etp,), f32),
                  jax.ShapeDtypeStruct((etp,), f32),
                  jax.ShapeDtypeStruct((nb,), f32)],
        mesh=mesh,
        scratch_types=[pltpu.VMEM((nb,), f32),
                       pltpu.VMEM((nb,), f32),
                       pltpu.VMEM((ch,), i32),
                       pltpu.VMEM((ch,), i32),
                       pltpu.VMEM((ch,), f32),
                       pltpu.VMEM((ch,), f32),
                       pltpu.VMEM((ch,), f32),
                       pltpu.VMEM((ch,), i32),
                       pltpu.VMEM((ch,), f32),
                       pltpu.VMEM((ch,), f32),
                       pltpu.VMEM((ch,), f32),
                       pltpu.VMEM((ch,), f32),
                       pltpu.VMEM((ch,), f32),
                       pltpu.VMEM((ch,), i32),
                       pltpu.VMEM((_L,), f32),
                       pltpu.SemaphoreType.DMA],
        compiler_params=pltpu.CompilerParams(needs_layout_passes=False),
    )
    def k(base_hbm, stot_hbm, col_hbm, row_hbm, ew_hbm, em_hbm, x_hbm,
          kc_hbm, fill_hbm,
          rows_hbm, cols_hbm, ewo_hbm, emo_hbm, y_hbm, intra_hbm,
          base_v, s_v, col_v, row_v, ew_v, em_v, x_v, pos_v, p_v, kc_v,
          y_v, ewo_v, emo_v, ii_v, fill_v, sem):
        wid = lax.axis_index("s") * 2 + lax.axis_index("c")
        e0 = pl.multiple_of(wid * ch, 8)
        pltpu.sync_copy(base_hbm.at[wid], base_v)
        pltpu.sync_copy(stot_hbm, s_v)
        pltpu.sync_copy(col_hbm.at[pl.ds(e0, ch)], col_v)
        pltpu.sync_copy(row_hbm.at[pl.ds(e0, ch)], row_v)
        pltpu.sync_copy(ew_hbm.at[pl.ds(e0, ch)], ew_v)
        pltpu.sync_copy(em_hbm.at[pl.ds(e0, ch)], em_v)
        pltpu.sync_copy(x_hbm.at[pl.ds(e0, ch)], x_v)
        pltpu.sync_copy(fill_hbm, fill_v)

        def gbody(g, _):
            sl = pl.ds(pl.multiple_of(g * _L, _L), _L)
            c = col_v[sl]
            bg = plsc.load_gather(base_v, [c])
            cnt, lastm = plsc.scan_count(c)
            cntf = cnt.astype(f32)
            plsc.store_scatter(base_v, [c], bg + cntf, mask=lastm)
            pos_v[sl] = (bg + cntf - 1.0).astype(i32)
            sg = plsc.load_gather(s_v, [c])
            p = x_v[sl] / (sg + 1e-16)
            p_v[sl] = jnp.minimum(jnp.maximum(p, 1e-6), 1.0 - 1e-6)
            return ()

        lax.fori_loop(0, ch // _L, gbody, (), unroll=2)

        pltpu.async_copy(kc_hbm.at[pos_v], kc_v, sem).wait()

        fillv = fill_v[...]
        nvec = jnp.full((_L,), n, i32)

        def fbody(g, _):
            sl = pl.ds(pl.multiple_of(g * _L, _L), _L)
            p = p_v[sl]
            q = 1.0 - p
            aa = p * p
            bb = kc_v[sl] * (q * q)
            y = aa / (aa + bb)
            yh = jnp.where(y > 0.5, 1.0, 0.0)
            yst = (yh - y) + y
            ew = ew_v[sl]
            em = em_v[sl]
            y_v[sl] = y
            ewo_v[sl] = jnp.where(ew == 0.0, yst, ew)
            emo_v[sl] = jnp.where((em == 0.0) & (yh == 1.0), fillv, em)
            ii_v[sl] = jnp.where(em == -1.0, col_v[sl], nvec)
            return ()

        lax.fori_loop(0, ch // _L, fbody, (), unroll=2)

        pltpu.sync_copy(row_v, rows_hbm.at[pos_v])
        pltpu.sync_copy(col_v, cols_hbm.at[pos_v])
        pltpu.sync_copy(ewo_v, ewo_hbm.at[pos_v])
        pltpu.sync_copy(emo_v, emo_hbm.at[pos_v])
        pltpu.sync_copy(y_v, y_hbm.at[pos_v])
        pltpu.sync_copy(y_v, intra_hbm.at[ii_v])

    return k(base, stot, col, row, ew, em, expw, kc, fill)


def kernel(x, edge_index, edge_weight, edge_mask, layer, att):
    n, d = x.shape
    e = edge_index.shape[1]
    et = e + n
    ch = _ceil_to(et, _NW * _L) // _NW
    etp = ch * _NW
    nb = _ceil_to(n + 1, 1024)

    idt = edge_index.dtype
    loop = jnp.arange(n, dtype=idt)
    pad = etp - et
    row = jnp.concatenate([edge_index[0], loop,
                           jnp.zeros((pad,), idt)]).astype(i32)
    col = jnp.concatenate([edge_index[1], loop,
                           jnp.full((pad,), n, idt)]).astype(i32)
    ew = jnp.concatenate([edge_weight, jnp.zeros((n,), f32),
                          jnp.zeros((pad,), f32)])
    em = jnp.concatenate([edge_mask, jnp.full((n,), -1.0, f32),
                          jnp.zeros((pad,), f32)])

    w8 = jnp.zeros((d, 8), f32)
    w8 = w8.at[:, 0].set(att[0, :d]).at[:, 1].set(att[0, d:])
    ab = _k1_matvec(x, w8)
    a = jnp.concatenate([ab[:, 0], jnp.zeros((nb - n,), f32)])
    b = jnp.concatenate([ab[:, 1], jnp.zeros((nb - n,), f32)])

    u = jax.random.uniform(jax.random.key(42), (et,),
                           minval=1e-6, maxval=1.0 - 1e-6)
    kc = jnp.concatenate([((1.0 - u) / u) ** 2, jnp.ones((pad,), f32)])

    h, s, expw = _k2_scores(a, b, col, row, ew, nb=nb, ch=ch, etp=etp)
    base, stot, _ = _k3_offsets(h, s, nb=nb)
    fill = jnp.full((_L,), jnp.asarray(layer + 1).astype(f32))
    rows, cols, ewo, emo, ysoft, intra = _k4_scatter(
        base, stot, col, row, ew, em, expw, kc, fill,
        nb=nb, ch=ch, etp=etp, n=n)

    edge_index_out = jnp.stack([rows[:et], cols[:et]]).astype(idt)
    return (edge_index_out, ewo[:et], ysoft[:et], emo[:et], intra[:n])
